# baseline (speedup 1.0000x reference)
import functools

import jax
import jax.numpy as jnp
from jax import lax
from jax.experimental import pallas as pl
from jax.experimental.pallas import tpu as pltpu
from jax.experimental.pallas import tpu_sc as plsc

N = 10000
E = 320000
D = 128
H = 64
L = 32

NC = 2
NS = 16
NW = NC * NS
EPW = E // NW
CH = 80
NBUF = 5
CPW = EPW // CH
GROUPS = CPW // NBUF
NBLK = N // CH
JMAX = -(-NBLK // NS)

_f32 = jnp.float32


def _seg_mesh_kernel(with_deg):
    mesh = plsc.VectorSubcoreMesh(core_axis_name="c", subcore_axis_name="s")
    out_type = [jax.ShapeDtypeStruct((NC, N, H), _f32)]
    scratch = [
        pltpu.VMEM_SHARED((N, H), _f32),
        pltpu.VMEM((3, NBUF, CH), jnp.int32),
        pltpu.VMEM((3, NBUF, CH), jnp.int32),
        pltpu.VMEM((3, NBUF, CH, H), _f32),
        pltpu.SemaphoreType.DMA,
        pltpu.SemaphoreType.DMA,
        pltpu.SemaphoreType.DMA,
        pltpu.SemaphoreType.DMA,
        pltpu.SemaphoreType.DMA,
        pltpu.SemaphoreType.DMA,
        pltpu.SemaphoreType.DMA,
    ]
    if with_deg:
        out_type.append(jax.ShapeDtypeStruct((NC, N, 16), _f32))
        scratch += [
            pltpu.VMEM_SHARED((N, 16), _f32),
            pltpu.VMEM((CH, 16), _f32),
        ]

    @functools.partial(
        pl.kernel,
        out_type=tuple(out_type) if with_deg else out_type[0],
        mesh=mesh, scratch_types=tuple(scratch),
        compiler_params=pltpu.CompilerParams(use_tc_tiling_on_sc=False))
    def k(table_r, esrc_r, edst_r, z64_r, z16_r, ones_r, *rest):
        if with_deg:
            (out_r, dout_r, acc, srcb, dstb, rows, gs0, gs1, gs2,
             ss0, ss1, ss2, isem, dega, ones_v) = rest
        else:
            (out_r, acc, srcb, dstb, rows, gs0, gs1, gs2,
             ss0, ss1, ss2, isem) = rest
        gsems = (gs0, gs1, gs2)
        ssems = (ss0, ss1, ss2)
        cid = lax.axis_index("c")
        sid = lax.axis_index("s")
        wid = sid * NC + cid

        def init_copies(j):
            blk = sid + NS * j
            cps = [pltpu.make_async_copy(z64_r.at[pl.ds(blk * CH, CH)],
                                         acc.at[pl.ds(blk * CH, CH)], isem)]
            if with_deg:
                cps.append(pltpu.make_async_copy(
                    z16_r.at[pl.ds(blk * CH, CH)],
                    dega.at[pl.ds(blk * CH, CH)], isem))
            return blk, cps

        def init_issue(j, carry):
            blk, cps = init_copies(j)

            @pl.when(blk < NBLK)
            def _():
                for c in cps:
                    c.start()

            return carry

        def init_drain(j, carry):
            blk, cps = init_copies(j)

            @pl.when(blk < NBLK)
            def _():
                for c in cps:
                    c.wait()

            return carry

        lax.fori_loop(0, JMAX, init_issue, 0)
        if with_deg:
            pltpu.sync_copy(ones_r, ones_v)
        lax.fori_loop(0, JMAX, init_drain, 0)
        plsc.subcore_barrier()

        def idx_copies(g, s):
            grow = wid * GROUPS + g
            return (
                pltpu.make_async_copy(esrc_r.at[grow], srcb.at[s], isem),
                pltpu.make_async_copy(edst_r.at[grow], dstb.at[s], isem),
            )

        def gather_copies(s):
            return [pltpu.make_async_copy(table_r.at[srcb.at[s, b]],
                                          rows.at[s, b], gsems[s])
                    for b in range(NBUF)]

        def scatter_copies(s):
            cps = []
            for b in range(NBUF):
                cps.append(pltpu.make_async_copy(
                    rows.at[s, b], acc.at[dstb.at[s, b]], ssems[s]))
                if with_deg:
                    cps.append(pltpu.make_async_copy(
                        ones_v, dega.at[dstb.at[s, b]], ssems[s]))
            return cps

        def issue_scatters(s):
            for b in range(NBUF):
                pltpu.async_copy(rows.at[s, b],
                                 acc.at[dstb.at[s, b]], ssems[s], add=True)
                if with_deg:
                    pltpu.async_copy(ones_v,
                                     dega.at[dstb.at[s, b]], ssems[s],
                                     add=True)

        for g0 in (0, 1):
            for c in idx_copies(g0, g0):
                c.start()
        for g0 in (0, 1):
            for c in idx_copies(g0, g0):
                c.wait()
            for c in gather_copies(g0):
                c.start()

        def step(g, j):
            sp = (j + 2) % 3
            for c in gather_copies(j):
                c.wait()

            @pl.when(g > 0)
            def _():
                for c in scatter_copies(sp):
                    c.wait()

            @pl.when(g < GROUPS - 2)
            def _():
                for c in idx_copies(g + 2, sp):
                    c.start()

            issue_scatters(j)

            @pl.when(g < GROUPS - 2)
            def _():
                for c in idx_copies(g + 2, sp):
                    c.wait()
                for c in gather_copies(sp):
                    c.start()

        def ubody(u, carry):
            for j in range(3):
                step(3 * u + j, j)
            return carry

        lax.fori_loop(0, (GROUPS - 1) // 3, ubody, 0)

        for c in gather_copies(0):
            c.wait()
        for c in scatter_copies(2):
            c.wait()
        issue_scatters(0)
        for c in scatter_copies(0):
            c.wait()
        plsc.subcore_barrier()

        def wb_copies(j):
            blk = sid + NS * j
            cps = [pltpu.make_async_copy(
                acc.at[pl.ds(blk * CH, CH)],
                out_r.at[cid, pl.ds(blk * CH, CH)], isem)]
            if with_deg:
                cps.append(pltpu.make_async_copy(
                    dega.at[pl.ds(blk * CH, CH)],
                    dout_r.at[cid, pl.ds(blk * CH, CH)], isem))
            return blk, cps

        def wb_issue(j, carry):
            blk, cps = wb_copies(j)

            @pl.when(blk < NBLK)
            def _():
                for c in cps:
                    c.start()

            return carry

        def wb_drain(j, carry):
            blk, cps = wb_copies(j)

            @pl.when(blk < NBLK)
            def _():
                for c in cps:
                    c.wait()

            return carry

        lax.fori_loop(0, JMAX, wb_issue, 0)
        lax.fori_loop(0, JMAX, wb_drain, 0)

    return k


_seg_with_deg = _seg_mesh_kernel(True)
_seg_plain = _seg_mesh_kernel(False)


PB = NBUF * CH
NG16 = PB // 16


def _decode_dot():
    mesh = plsc.VectorSubcoreMesh(core_axis_name="c", subcore_axis_name="s")
    scratch = [
        pltpu.VMEM((3, NBUF, CH), jnp.int32),
        pltpu.VMEM((3, NBUF, CH), jnp.int32),
        pltpu.VMEM((3 * NBUF * CH, L), _f32),
        pltpu.VMEM((3 * NBUF * CH, L), _f32),
        pltpu.VMEM((3, PB), _f32),
        pltpu.SemaphoreType.DMA,
        pltpu.SemaphoreType.DMA,
        pltpu.SemaphoreType.DMA,
        pltpu.SemaphoreType.DMA,
        pltpu.SemaphoreType.DMA,
        pltpu.SemaphoreType.DMA,
        pltpu.SemaphoreType.DMA,
    ]

    @functools.partial(
        pl.kernel,
        out_type=jax.ShapeDtypeStruct((E,), _f32),
        mesh=mesh, scratch_types=tuple(scratch),
        compiler_params=pltpu.CompilerParams(use_tc_tiling_on_sc=False,
                                             needs_layout_passes=False))
    def k(mu_r, src_r, dst_r, out_r, srcb, dstb, arows, brows, outv,
          gs0, gs1, gs2, ws0, ws1, ws2, isem):
        gsems = (gs0, gs1, gs2)
        wsems = (ws0, ws1, ws2)
        cid = lax.axis_index("c")
        sid = lax.axis_index("s")
        wid = sid * NC + cid

        def idx_copies(g, s):
            grow = wid * GROUPS + g
            return (
                pltpu.make_async_copy(src_r.at[grow], srcb.at[s], isem),
                pltpu.make_async_copy(dst_r.at[grow], dstb.at[s], isem),
            )

        def gather_copies(s):
            cps = []
            for b in range(NBUF):
                base = (s * NBUF + b) * CH
                cps.append(pltpu.make_async_copy(
                    mu_r.at[srcb.at[s, b]], arows.at[pl.ds(base, CH)],
                    gsems[s]))
                cps.append(pltpu.make_async_copy(
                    mu_r.at[dstb.at[s, b]], brows.at[pl.ds(base, CH)],
                    gsems[s]))
            return cps

        def write_copy(g, s):
            return pltpu.make_async_copy(
                outv.at[s], out_r.at[pl.ds(wid * EPW + g * PB, PB)],
                wsems[s])

        def compute_group(s):
            lane = lax.iota(jnp.int32, 16)

            def gbody(i, carry):
                rbase = s * (NBUF * CH) + i * 16
                res = jnp.zeros((16,), _f32)
                for e in range(16):
                    row = rbase + e
                    va0 = arows[row, pl.ds(0, 16)]
                    va1 = arows[row, pl.ds(16, 16)]
                    vb0 = brows[row, pl.ds(0, 16)]
                    vb1 = brows[row, pl.ds(16, 16)]
                    t = jnp.sum(va0 * vb0 + va1 * vb1)
                    res = jnp.where(lane == e, t, res)
                outv[s, pl.ds(i * 16, 16)] = res
                return carry

            lax.fori_loop(0, NG16, gbody, 0)

            def sbody(j, carry):
                v = outv[s, pl.ds(j * 16, 16)]
                outv[s, pl.ds(j * 16, 16)] = 1.0 / (1.0 + jnp.exp(-v))
                return carry

            lax.fori_loop(0, NG16, sbody, 0)

        for g0 in (0, 1):
            for c in idx_copies(g0, g0):
                c.start()
        for g0 in (0, 1):
            for c in idx_copies(g0, g0):
                c.wait()
            for c in gather_copies(g0):
                c.start()

        def step(g, j):
            sp = (j + 2) % 3

            @pl.when(g < GROUPS - 2)
            def _():
                for c in idx_copies(g + 2, sp):
                    c.start()

            for c in gather_copies(j):
                c.wait()

            @pl.when(g < GROUPS - 2)
            def _():
                for c in idx_copies(g + 2, sp):
                    c.wait()
                for c in gather_copies(sp):
                    c.start()

            @pl.when(g >= 3)
            def _():
                write_copy(g - 3, j).wait()

            compute_group(j)
            write_copy(g, j).start()

        def ubody(u, carry):
            for j in range(3):
                step(3 * u + j, j)
            return carry

        lax.fori_loop(0, (GROUPS - 1) // 3, ubody, 0)

        g_last = GROUPS - 1
        for c in gather_copies(0):
            c.wait()
        write_copy(g_last - 3, 0).wait()
        compute_group(0)
        write_copy(g_last, 0).start()
        write_copy(g_last - 2, 1).wait()
        write_copy(g_last - 1, 2).wait()
        write_copy(g_last, 0).wait()

    return k


_decode = _decode_dot()


def _tc_project(x, Wl, Wr, b):
    Bn = 2000
    d_in = x.shape[1]

    def body(xr, wl, wr, br, yr, rr):
        xb = xr[...]
        yr[...] = jnp.dot(xb, wl[...], preferred_element_type=_f32)
        rr[...] = jnp.dot(xb, wr[...], preferred_element_type=_f32) + br[...][None, :]

    return pl.pallas_call(
        body,
        grid=(N // Bn,),
        in_specs=[
            pl.BlockSpec((Bn, d_in), lambda i: (i, 0)),
            pl.BlockSpec((d_in, H), lambda i: (0, 0)),
            pl.BlockSpec((d_in, H), lambda i: (0, 0)),
            pl.BlockSpec((H,), lambda i: (0,)),
        ],
        out_specs=[pl.BlockSpec((Bn, H), lambda i: (i, 0)),
                   pl.BlockSpec((Bn, H), lambda i: (i, 0))],
        out_shape=[jax.ShapeDtypeStruct((N, H), _f32)] * 2,
    )(x, Wl, Wr, b)


def _tc_norm_project(p0, p1, d0, d1, r, Wl, Wr, b):
    Bn = 2000

    def body(p0r, p1r, d0r, d1r, rr, wl, wr, br, yo, ro):
        deg = d0r[...][:, 0] + d1r[...][:, 0]
        dmax = jnp.maximum(deg, 1.0)
        h = jnp.maximum((p0r[...] + p1r[...]) / dmax[:, None] + rr[...], 0.0)
        yo[...] = jnp.dot(h, wl[...], preferred_element_type=_f32)
        ro[...] = jnp.dot(h, wr[...], preferred_element_type=_f32) + br[...][None, :]

    return pl.pallas_call(
        body,
        grid=(N // Bn,),
        in_specs=[
            pl.BlockSpec((Bn, H), lambda i: (i, 0)),
            pl.BlockSpec((Bn, H), lambda i: (i, 0)),
            pl.BlockSpec((Bn, 16), lambda i: (i, 0)),
            pl.BlockSpec((Bn, 16), lambda i: (i, 0)),
            pl.BlockSpec((Bn, H), lambda i: (i, 0)),
            pl.BlockSpec((H, H), lambda i: (0, 0)),
            pl.BlockSpec((H, H), lambda i: (0, 0)),
            pl.BlockSpec((H,), lambda i: (0,)),
        ],
        out_specs=[pl.BlockSpec((Bn, H), lambda i: (i, 0)),
                   pl.BlockSpec((Bn, H), lambda i: (i, 0))],
        out_shape=[jax.ShapeDtypeStruct((N, H), _f32)] * 2,
    )(p0, p1, d0, d1, r, Wl, Wr, b)


def _tc_norm_mu(p0, p1, d0, d1, r, Wmu, bmu):
    Bn = 2000

    def body(p0r, p1r, d0r, d1r, rr, wm, bm, muo):
        deg = d0r[...][:, 0] + d1r[...][:, 0]
        dmax = jnp.maximum(deg, 1.0)
        h = jnp.maximum((p0r[...] + p1r[...]) / dmax[:, None] + rr[...], 0.0)
        muo[...] = jnp.dot(h, wm[...], preferred_element_type=_f32) + bm[...][None, :]

    return pl.pallas_call(
        body,
        grid=(N // Bn,),
        in_specs=[
            pl.BlockSpec((Bn, H), lambda i: (i, 0)),
            pl.BlockSpec((Bn, H), lambda i: (i, 0)),
            pl.BlockSpec((Bn, 16), lambda i: (i, 0)),
            pl.BlockSpec((Bn, 16), lambda i: (i, 0)),
            pl.BlockSpec((Bn, H), lambda i: (i, 0)),
            pl.BlockSpec((H, L), lambda i: (0, 0)),
            pl.BlockSpec((L,), lambda i: (0,)),
        ],
        out_specs=pl.BlockSpec((Bn, L), lambda i: (i, 0)),
        out_shape=jax.ShapeDtypeStruct((N, L), _f32),
    )(p0, p1, d0, d1, r, Wmu, bmu)


def kernel(x, edge_index, src, dst, Wl1, Wr1, b1, Wl2, Wr2, b2, Wmu, bmu):
    esrc2 = edge_index[0].reshape(NW * GROUPS, NBUF, CH)
    edst2 = edge_index[1].reshape(NW * GROUPS, NBUF, CH)
    src2 = src.reshape(NW * GROUPS, NBUF, CH)
    dst2 = dst.reshape(NW * GROUPS, NBUF, CH)
    z64 = jnp.zeros((N, H), _f32)
    z16 = jnp.zeros((N, 16), _f32)
    ones16 = jnp.ones((CH, 16), _f32)

    y1, r1 = _tc_project(x, Wl1, Wr1, b1)
    parts1, degp = _seg_with_deg(y1, esrc2, edst2, z64, z16, ones16)
    y2, r2 = _tc_norm_project(parts1[0], parts1[1], degp[0], degp[1], r1,
                              Wl2, Wr2, b2)
    parts2 = _seg_plain(y2, esrc2, edst2, z64, z16, ones16)
    mu = _tc_norm_mu(parts2[0], parts2[1], degp[0], degp[1], r2, Wmu, bmu)
    return _decode(mu, src2, dst2)


# speedup vs baseline: 1.0183x; 1.0183x over previous
import functools

import jax
import jax.numpy as jnp
from jax import lax
from jax.experimental import pallas as pl
from jax.experimental.pallas import tpu as pltpu
from jax.experimental.pallas import tpu_sc as plsc

N = 10000
E = 320000
D = 128
H = 64
L = 32

NC = 2
NS = 16
NW = NC * NS
EPW = E // NW
CH = 80
NBUF = 5
CPW = EPW // CH
GROUPS = CPW // NBUF
NBLK = N // CH
JMAX = -(-NBLK // NS)

_f32 = jnp.float32


def _seg_mesh_kernel(with_deg):
    mesh = plsc.VectorSubcoreMesh(core_axis_name="c", subcore_axis_name="s")
    out_type = [jax.ShapeDtypeStruct((NC, N, H), _f32)]
    scratch = [
        pltpu.VMEM_SHARED((N, H), _f32),
        pltpu.VMEM((3, NBUF, CH), jnp.int32),
        pltpu.VMEM((3, NBUF, CH), jnp.int32),
        pltpu.VMEM((3, NBUF, CH, H), _f32),
        pltpu.SemaphoreType.DMA,
        pltpu.SemaphoreType.DMA,
        pltpu.SemaphoreType.DMA,
        pltpu.SemaphoreType.DMA,
        pltpu.SemaphoreType.DMA,
        pltpu.SemaphoreType.DMA,
        pltpu.SemaphoreType.DMA,
        pltpu.SemaphoreType.DMA,
    ]
    if with_deg:
        out_type.append(jax.ShapeDtypeStruct((NC, N, 16), _f32))
        scratch += [
            pltpu.VMEM_SHARED((N, 16), _f32),
            pltpu.VMEM((CH, 16), _f32),
        ]

    @functools.partial(
        pl.kernel,
        out_type=tuple(out_type) if with_deg else out_type[0],
        mesh=mesh, scratch_types=tuple(scratch),
        compiler_params=pltpu.CompilerParams(use_tc_tiling_on_sc=False))
    def k(table_r, esrc_r, edst_r, z64_r, z16_r, ones_r, *rest):
        if with_deg:
            (out_r, dout_r, acc, srcb, dstb, rows, gs0, gs1, gs2,
             ss0, ss1, ss2, isem, zsem, dega, ones_v) = rest
        else:
            (out_r, acc, srcb, dstb, rows, gs0, gs1, gs2,
             ss0, ss1, ss2, isem, zsem) = rest
        gsems = (gs0, gs1, gs2)
        ssems = (ss0, ss1, ss2)
        cid = lax.axis_index("c")
        sid = lax.axis_index("s")
        wid = sid * NC + cid

        def init_copies(j):
            blk = sid + NS * j
            cps = [pltpu.make_async_copy(z64_r.at[pl.ds(blk * CH, CH)],
                                         acc.at[pl.ds(blk * CH, CH)], zsem)]
            if with_deg:
                cps.append(pltpu.make_async_copy(
                    z16_r.at[pl.ds(blk * CH, CH)],
                    dega.at[pl.ds(blk * CH, CH)], zsem))
            return blk, cps

        def init_issue(j, carry):
            blk, cps = init_copies(j)

            @pl.when(blk < NBLK)
            def _():
                for c in cps:
                    c.start()

            return carry

        def init_drain(j, carry):
            blk, cps = init_copies(j)

            @pl.when(blk < NBLK)
            def _():
                for c in cps:
                    c.wait()

            return carry

        lax.fori_loop(0, JMAX, init_issue, 0)
        if with_deg:
            pltpu.sync_copy(ones_r, ones_v)

        def idx_copies(g, s):
            grow = wid * GROUPS + g
            return (
                pltpu.make_async_copy(esrc_r.at[grow], srcb.at[s], isem),
                pltpu.make_async_copy(edst_r.at[grow], dstb.at[s], isem),
            )

        def gather_copies(s):
            return [pltpu.make_async_copy(table_r.at[srcb.at[s, b]],
                                          rows.at[s, b], gsems[s])
                    for b in range(NBUF)]

        def scatter_copies(s):
            cps = []
            for b in range(NBUF):
                cps.append(pltpu.make_async_copy(
                    rows.at[s, b], acc.at[dstb.at[s, b]], ssems[s]))
                if with_deg:
                    cps.append(pltpu.make_async_copy(
                        ones_v, dega.at[dstb.at[s, b]], ssems[s]))
            return cps

        def issue_scatters(s):
            for b in range(NBUF):
                pltpu.async_copy(rows.at[s, b],
                                 acc.at[dstb.at[s, b]], ssems[s], add=True)
                if with_deg:
                    pltpu.async_copy(ones_v,
                                     dega.at[dstb.at[s, b]], ssems[s],
                                     add=True)

        for g0 in (0, 1):
            for c in idx_copies(g0, g0):
                c.start()
        for g0 in (0, 1):
            for c in idx_copies(g0, g0):
                c.wait()
            for c in gather_copies(g0):
                c.start()
        lax.fori_loop(0, JMAX, init_drain, 0)
        plsc.subcore_barrier()

        def step(g, j):
            sp = (j + 2) % 3
            for c in gather_copies(j):
                c.wait()

            @pl.when(g > 0)
            def _():
                for c in scatter_copies(sp):
                    c.wait()

            @pl.when(g < GROUPS - 2)
            def _():
                for c in idx_copies(g + 2, sp):
                    c.start()

            issue_scatters(j)

            @pl.when(g < GROUPS - 2)
            def _():
                for c in idx_copies(g + 2, sp):
                    c.wait()
                for c in gather_copies(sp):
                    c.start()

        def ubody(u, carry):
            for j in range(3):
                step(3 * u + j, j)
            return carry

        lax.fori_loop(0, (GROUPS - 1) // 3, ubody, 0)

        for c in gather_copies(0):
            c.wait()
        for c in scatter_copies(2):
            c.wait()
        issue_scatters(0)
        for c in scatter_copies(0):
            c.wait()
        plsc.subcore_barrier()

        def wb_copies(j):
            blk = sid + NS * j
            cps = [pltpu.make_async_copy(
                acc.at[pl.ds(blk * CH, CH)],
                out_r.at[cid, pl.ds(blk * CH, CH)], zsem)]
            if with_deg:
                cps.append(pltpu.make_async_copy(
                    dega.at[pl.ds(blk * CH, CH)],
                    dout_r.at[cid, pl.ds(blk * CH, CH)], zsem))
            return blk, cps

        def wb_issue(j, carry):
            blk, cps = wb_copies(j)

            @pl.when(blk < NBLK)
            def _():
                for c in cps:
                    c.start()

            return carry

        def wb_drain(j, carry):
            blk, cps = wb_copies(j)

            @pl.when(blk < NBLK)
            def _():
                for c in cps:
                    c.wait()

            return carry

        lax.fori_loop(0, JMAX, wb_issue, 0)
        lax.fori_loop(0, JMAX, wb_drain, 0)

    return k


_seg_with_deg = _seg_mesh_kernel(True)
_seg_plain = _seg_mesh_kernel(False)


PB = NBUF * CH
NG16 = PB // 16


def _decode_dot():
    mesh = plsc.VectorSubcoreMesh(core_axis_name="c", subcore_axis_name="s")
    scratch = [
        pltpu.VMEM((3, NBUF, CH), jnp.int32),
        pltpu.VMEM((3, NBUF, CH), jnp.int32),
        pltpu.VMEM((3 * NBUF * CH, L), _f32),
        pltpu.VMEM((3 * NBUF * CH, L), _f32),
        pltpu.VMEM((3, PB), _f32),
        pltpu.SemaphoreType.DMA,
        pltpu.SemaphoreType.DMA,
        pltpu.SemaphoreType.DMA,
        pltpu.SemaphoreType.DMA,
        pltpu.SemaphoreType.DMA,
        pltpu.SemaphoreType.DMA,
        pltpu.SemaphoreType.DMA,
    ]

    @functools.partial(
        pl.kernel,
        out_type=jax.ShapeDtypeStruct((E,), _f32),
        mesh=mesh, scratch_types=tuple(scratch),
        compiler_params=pltpu.CompilerParams(use_tc_tiling_on_sc=False,
                                             needs_layout_passes=False))
    def k(mu_r, src_r, dst_r, out_r, srcb, dstb, arows, brows, outv,
          gs0, gs1, gs2, ws0, ws1, ws2, isem):
        gsems = (gs0, gs1, gs2)
        wsems = (ws0, ws1, ws2)
        cid = lax.axis_index("c")
        sid = lax.axis_index("s")
        wid = sid * NC + cid

        def idx_copies(g, s):
            grow = wid * GROUPS + g
            return (
                pltpu.make_async_copy(src_r.at[grow], srcb.at[s], isem),
                pltpu.make_async_copy(dst_r.at[grow], dstb.at[s], isem),
            )

        def gather_copies(s):
            cps = []
            for b in range(NBUF):
                base = (s * NBUF + b) * CH
                cps.append(pltpu.make_async_copy(
                    mu_r.at[srcb.at[s, b]], arows.at[pl.ds(base, CH)],
                    gsems[s]))
                cps.append(pltpu.make_async_copy(
                    mu_r.at[dstb.at[s, b]], brows.at[pl.ds(base, CH)],
                    gsems[s]))
            return cps

        def write_copy(g, s):
            return pltpu.make_async_copy(
                outv.at[s], out_r.at[pl.ds(wid * EPW + g * PB, PB)],
                wsems[s])

        def compute_group(s):
            lane = lax.iota(jnp.int32, 16)

            def gbody(i, carry):
                rbase = s * (NBUF * CH) + i * 16
                res = jnp.zeros((16,), _f32)
                for e in range(16):
                    row = rbase + e
                    va0 = arows[row, pl.ds(0, 16)]
                    va1 = arows[row, pl.ds(16, 16)]
                    vb0 = brows[row, pl.ds(0, 16)]
                    vb1 = brows[row, pl.ds(16, 16)]
                    t = jnp.sum(va0 * vb0 + va1 * vb1)
                    res = jnp.where(lane == e, t, res)
                outv[s, pl.ds(i * 16, 16)] = 1.0 / (1.0 + jnp.exp(-res))
                return carry

            lax.fori_loop(0, NG16, gbody, 0)

        for g0 in (0, 1):
            for c in idx_copies(g0, g0):
                c.start()
        for g0 in (0, 1):
            for c in idx_copies(g0, g0):
                c.wait()
            for c in gather_copies(g0):
                c.start()

        def step(g, j):
            sp = (j + 2) % 3

            @pl.when(g < GROUPS - 2)
            def _():
                for c in idx_copies(g + 2, sp):
                    c.start()

            for c in gather_copies(j):
                c.wait()

            @pl.when(g < GROUPS - 2)
            def _():
                for c in idx_copies(g + 2, sp):
                    c.wait()
                for c in gather_copies(sp):
                    c.start()

            @pl.when(g >= 3)
            def _():
                write_copy(g - 3, j).wait()

            compute_group(j)
            write_copy(g, j).start()

        def ubody(u, carry):
            for j in range(3):
                step(3 * u + j, j)
            return carry

        lax.fori_loop(0, (GROUPS - 1) // 3, ubody, 0)

        g_last = GROUPS - 1
        for c in gather_copies(0):
            c.wait()
        write_copy(g_last - 3, 0).wait()
        compute_group(0)
        write_copy(g_last, 0).start()
        write_copy(g_last - 2, 1).wait()
        write_copy(g_last - 1, 2).wait()
        write_copy(g_last, 0).wait()

    return k


_decode = _decode_dot()


def _tc_project(x, Wl, Wr, b):
    Bn = 2000
    d_in = x.shape[1]

    def body(xr, wl, wr, br, yr, rr):
        xb = xr[...]
        yr[...] = jnp.dot(xb, wl[...], preferred_element_type=_f32)
        rr[...] = jnp.dot(xb, wr[...], preferred_element_type=_f32) + br[...][None, :]

    return pl.pallas_call(
        body,
        grid=(N // Bn,),
        in_specs=[
            pl.BlockSpec((Bn, d_in), lambda i: (i, 0)),
            pl.BlockSpec((d_in, H), lambda i: (0, 0)),
            pl.BlockSpec((d_in, H), lambda i: (0, 0)),
            pl.BlockSpec((H,), lambda i: (0,)),
        ],
        out_specs=[pl.BlockSpec((Bn, H), lambda i: (i, 0)),
                   pl.BlockSpec((Bn, H), lambda i: (i, 0))],
        out_shape=[jax.ShapeDtypeStruct((N, H), _f32)] * 2,
    )(x, Wl, Wr, b)


def _tc_norm_project(p0, p1, d0, d1, r, Wl, Wr, b):
    Bn = 2000

    def body(p0r, p1r, d0r, d1r, rr, wl, wr, br, yo, ro):
        deg = d0r[...][:, 0] + d1r[...][:, 0]
        dmax = jnp.maximum(deg, 1.0)
        h = jnp.maximum((p0r[...] + p1r[...]) / dmax[:, None] + rr[...], 0.0)
        yo[...] = jnp.dot(h, wl[...], preferred_element_type=_f32)
        ro[...] = jnp.dot(h, wr[...], preferred_element_type=_f32) + br[...][None, :]

    return pl.pallas_call(
        body,
        grid=(N // Bn,),
        in_specs=[
            pl.BlockSpec((Bn, H), lambda i: (i, 0)),
            pl.BlockSpec((Bn, H), lambda i: (i, 0)),
            pl.BlockSpec((Bn, 16), lambda i: (i, 0)),
            pl.BlockSpec((Bn, 16), lambda i: (i, 0)),
            pl.BlockSpec((Bn, H), lambda i: (i, 0)),
            pl.BlockSpec((H, H), lambda i: (0, 0)),
            pl.BlockSpec((H, H), lambda i: (0, 0)),
            pl.BlockSpec((H,), lambda i: (0,)),
        ],
        out_specs=[pl.BlockSpec((Bn, H), lambda i: (i, 0)),
                   pl.BlockSpec((Bn, H), lambda i: (i, 0))],
        out_shape=[jax.ShapeDtypeStruct((N, H), _f32)] * 2,
    )(p0, p1, d0, d1, r, Wl, Wr, b)


def _tc_norm_mu(p0, p1, d0, d1, r, Wmu, bmu):
    Bn = 2000

    def body(p0r, p1r, d0r, d1r, rr, wm, bm, muo):
        deg = d0r[...][:, 0] + d1r[...][:, 0]
        dmax = jnp.maximum(deg, 1.0)
        h = jnp.maximum((p0r[...] + p1r[...]) / dmax[:, None] + rr[...], 0.0)
        muo[...] = jnp.dot(h, wm[...], preferred_element_type=_f32) + bm[...][None, :]

    return pl.pallas_call(
        body,
        grid=(N // Bn,),
        in_specs=[
            pl.BlockSpec((Bn, H), lambda i: (i, 0)),
            pl.BlockSpec((Bn, H), lambda i: (i, 0)),
            pl.BlockSpec((Bn, 16), lambda i: (i, 0)),
            pl.BlockSpec((Bn, 16), lambda i: (i, 0)),
            pl.BlockSpec((Bn, H), lambda i: (i, 0)),
            pl.BlockSpec((H, L), lambda i: (0, 0)),
            pl.BlockSpec((L,), lambda i: (0,)),
        ],
        out_specs=pl.BlockSpec((Bn, L), lambda i: (i, 0)),
        out_shape=jax.ShapeDtypeStruct((N, L), _f32),
    )(p0, p1, d0, d1, r, Wmu, bmu)


def kernel(x, edge_index, src, dst, Wl1, Wr1, b1, Wl2, Wr2, b2, Wmu, bmu):
    esrc2 = edge_index[0].reshape(NW * GROUPS, NBUF, CH)
    edst2 = edge_index[1].reshape(NW * GROUPS, NBUF, CH)
    src2 = src.reshape(NW * GROUPS, NBUF, CH)
    dst2 = dst.reshape(NW * GROUPS, NBUF, CH)
    z64 = jnp.zeros((N, H), _f32)
    z16 = jnp.zeros((N, 16), _f32)
    ones16 = jnp.ones((CH, 16), _f32)

    y1, r1 = _tc_project(x, Wl1, Wr1, b1)
    parts1, degp = _seg_with_deg(y1, esrc2, edst2, z64, z16, ones16)
    y2, r2 = _tc_norm_project(parts1[0], parts1[1], degp[0], degp[1], r1,
                              Wl2, Wr2, b2)
    parts2 = _seg_plain(y2, esrc2, edst2, z64, z16, ones16)
    mu = _tc_norm_mu(parts2[0], parts2[1], degp[0], degp[1], r2, Wmu, bmu)
    return _decode(mu, src2, dst2)
